# baseline (speedup 1.0000x reference)
import functools
import math

import jax
import jax.numpy as jnp
from jax.experimental import pallas as pl
from jax.experimental.pallas import tpu as pltpu


_RCHUNK = 4
_NBUF = 8
_LAG = 4


def _rep_kernel(bank_hbm, q_hbm, phi_ref, scratch, in_sems, out_sems,
                *, rows, nchunk, nbuf, lag, b, m):
    col = jax.lax.broadcasted_iota(jnp.int32, (1, m), 1).astype(jnp.float32)
    phi_ref[...] = (2.0 * math.pi / m) * col

    def in_copy(c):
        slot = c % nbuf
        return pltpu.make_async_copy(
            bank_hbm.at[pl.ds(c * rows, rows), :],
            scratch.at[slot],
            in_sems.at[slot],
        )

    def out_copy(c, j):
        slot = c % nbuf
        return pltpu.make_async_copy(
            scratch.at[slot],
            q_hbm.at[j, pl.ds(c * rows, rows), :],
            out_sems.at[slot, j],
        )

    for c in range(min(lag, nchunk)):
        in_copy(c).start()

    unwaited = {}
    for c in range(nchunk):
        in_copy(c).wait()
        for j in range(b):
            out_copy(c, j).start()
        unwaited[c] = True
        r = c + lag
        if r < nchunk:
            prev = r - nbuf
            if prev >= 0 and prev in unwaited:
                for j in range(b):
                    out_copy(prev, j).wait()
                del unwaited[prev]
            in_copy(r).start()
    for c in sorted(unwaited):
        for j in range(b):
            out_copy(c, j).wait()


def kernel(key_embed, bank):
    b = key_embed.shape[0]
    m, dim = bank.shape
    rows = _RCHUNK
    nchunk = dim // rows
    bank_t = bank.T
    qt, phi2d = pl.pallas_call(
        functools.partial(_rep_kernel, rows=rows, nchunk=nchunk,
                          nbuf=_NBUF, lag=_LAG, b=b, m=m),
        in_specs=[pl.BlockSpec(memory_space=pl.ANY)],
        out_specs=[
            pl.BlockSpec(memory_space=pl.ANY),
            pl.BlockSpec(memory_space=pltpu.VMEM),
        ],
        out_shape=[
            jax.ShapeDtypeStruct((b, dim, m), jnp.float32),
            jax.ShapeDtypeStruct((1, m), jnp.float32),
        ],
        scratch_shapes=[
            pltpu.VMEM((_NBUF, _RCHUNK, m), jnp.float32),
            pltpu.SemaphoreType.DMA((_NBUF,)),
            pltpu.SemaphoreType.DMA((_NBUF, 4)),
        ],
    )(bank_t)
    q = qt.transpose(0, 2, 1)
    q_valid = jnp.ones((b, m), dtype=bool)
    return (q, q_valid, phi2d.reshape(m))


# speedup vs baseline: 1.0373x; 1.0373x over previous
import functools
import math

import jax
import jax.numpy as jnp
from jax.experimental import pallas as pl
from jax.experimental.pallas import tpu as pltpu


_RCHUNK = 16
_NBUF = 4
_LAG = 2


def _rep_kernel(bank_hbm, q_hbm, phi_ref, scratch, in_sems, out_sems,
                *, rows, nchunk, nbuf, lag, b, m):
    col = jax.lax.broadcasted_iota(jnp.int32, (1, m), 1).astype(jnp.float32)
    phi_ref[...] = (2.0 * math.pi / m) * col

    def in_copy(c):
        slot = c % nbuf
        return pltpu.make_async_copy(
            bank_hbm.at[pl.ds(c * rows, rows), :],
            scratch.at[slot],
            in_sems.at[slot],
        )

    def out_copy(c, j):
        slot = c % nbuf
        return pltpu.make_async_copy(
            scratch.at[slot],
            q_hbm.at[j, pl.ds(c * rows, rows), :],
            out_sems.at[slot, j],
        )

    for c in range(min(lag, nchunk)):
        in_copy(c).start()

    unwaited = {}
    for c in range(nchunk):
        in_copy(c).wait()
        for j in range(b):
            out_copy(c, j).start()
        unwaited[c] = True
        r = c + lag
        if r < nchunk:
            prev = r - nbuf
            if prev >= 0 and prev in unwaited:
                for j in range(b):
                    out_copy(prev, j).wait()
                del unwaited[prev]
            in_copy(r).start()
    for c in sorted(unwaited):
        for j in range(b):
            out_copy(c, j).wait()


def kernel(key_embed, bank):
    b = key_embed.shape[0]
    m, dim = bank.shape
    rows = _RCHUNK
    nchunk = dim // rows
    bank_t = bank.T
    qt, phi2d = pl.pallas_call(
        functools.partial(_rep_kernel, rows=rows, nchunk=nchunk,
                          nbuf=_NBUF, lag=_LAG, b=b, m=m),
        in_specs=[pl.BlockSpec(memory_space=pl.ANY)],
        out_specs=[
            pl.BlockSpec(memory_space=pl.ANY),
            pl.BlockSpec(memory_space=pltpu.VMEM),
        ],
        out_shape=[
            jax.ShapeDtypeStruct((b, dim, m), jnp.float32),
            jax.ShapeDtypeStruct((1, m), jnp.float32),
        ],
        scratch_shapes=[
            pltpu.VMEM((_NBUF, _RCHUNK, m), jnp.float32),
            pltpu.SemaphoreType.DMA((_NBUF,)),
            pltpu.SemaphoreType.DMA((_NBUF, 4)),
        ],
    )(bank_t)
    q = qt.transpose(0, 2, 1)
    q_valid = jnp.ones((b, m), dtype=bool)
    return (q, q_valid, phi2d.reshape(m))
